# Initial kernel scaffold; baseline (speedup 1.0000x reference)
import jax
import jax.numpy as jnp
from jax.experimental import pallas as pl


def kernel(l_cxt, r_cxt, l_lbl, r_lbl, cxt_table, lbl_table):
    raise NotImplementedError("write your pallas kernel here")


# speedup vs baseline: 9.4040x; 9.4040x over previous
import functools

import jax
import jax.numpy as jnp
from jax import lax
from jax.experimental import pallas as pl
from jax.experimental.pallas import tpu as pltpu
from jax.experimental.pallas import tpu_sc as plsc

V = 100001
D = 64
B = 4096
L = 50
NC, NS = 2, 16
NW = NC * NS
BPW = B // NW
EPC = 2
NCHUNK = BPW // EPC
ROWS = EPC * L


def _sc_dots(cxt_idx, lbl_idx, cxt_table, lbl_table):
    mesh = plsc.VectorSubcoreMesh(core_axis_name="c", subcore_axis_name="s")

    @functools.partial(
        pl.kernel,
        out_type=jax.ShapeDtypeStruct((NW, 2, BPW), jnp.float32),
        mesh=mesh,
        scratch_types=[
            pltpu.VMEM((2, NCHUNK, ROWS), jnp.int32),
            pltpu.VMEM((2, BPW), jnp.int32),
            pltpu.VMEM((2, BPW, D), jnp.float32),
            pltpu.VMEM((2, ROWS, D), jnp.float32),
            pltpu.VMEM((2, BPW), jnp.float32),
            pltpu.SemaphoreType.DMA,
            pltpu.SemaphoreType.DMA,
            pltpu.SemaphoreType.DMA,
        ],
        compiler_params=pltpu.CompilerParams(use_tc_tiling_on_sc=False),
    )
    def kern(cxt_idx_hbm, lbl_idx_hbm, cxt_tab_hbm, lbl_tab_hbm, out_hbm,
             idx_v, lidx_v, lrows_v, buf_v, out_v, sem0, sem1, sem_l):
        wid = lax.axis_index("s") * NC + lax.axis_index("c")
        sems = (sem0, sem1)

        pltpu.sync_copy(cxt_idx_hbm.at[wid], idx_v)
        pltpu.sync_copy(lbl_idx_hbm.at[wid], lidx_v)
        pltpu.async_copy(lbl_tab_hbm.at[lidx_v.at[0]], lrows_v.at[0], sem_l).wait()
        pltpu.async_copy(lbl_tab_hbm.at[lidx_v.at[1]], lrows_v.at[1], sem_l).wait()

        lanes = lax.iota(jnp.int32, 16)
        for s in range(2):
            for b in range(2):
                pltpu.async_copy(
                    cxt_tab_hbm.at[idx_v.at[s, b]], buf_v.at[b], sems[b])

            def group16(g, _, s=s):
                dvec = jnp.zeros((16,), jnp.float32)
                for b8 in range(8):
                    chunk = 8 * g + b8
                    slot = b8 % 2
                    pltpu.make_async_copy(
                        cxt_tab_hbm.at[idx_v.at[s, slot]], buf_v.at[slot],
                        sems[slot]).wait()
                    for e in range(EPC):
                        bb = chunk * EPC + e
                        lane = b8 * EPC + e
                        lbl = [lrows_v[s, bb, pl.ds(16 * c, 16)]
                               for c in range(4)]

                        def row_acc(l, acc, e=e, slot=slot, lbl=lbl):
                            q = e * L + l
                            return tuple(
                                acc[c] + buf_v[slot, q, pl.ds(16 * c, 16)]
                                * lbl[c]
                                for c in range(4))

                        z = jnp.zeros((16,), jnp.float32)
                        a = lax.fori_loop(0, L, row_acc, (z, z, z, z))
                        tot = (a[0] + a[1]) + (a[2] + a[3])
                        for sh in (8, 4, 2, 1):
                            tot = tot + tot.at[lanes ^ sh].get(
                                mode="promise_in_bounds")
                        dvec = jnp.where(lanes == lane, tot, dvec)
                    @pl.when(chunk + 2 < NCHUNK)
                    def _(slot=slot, chunk=chunk, s=s):
                        pltpu.async_copy(
                            cxt_tab_hbm.at[idx_v.at[s, chunk + 2]],
                            buf_v.at[slot], sems[slot])
                out_v[s, pl.ds(g * 16, 16)] = dvec
                return 0

            lax.fori_loop(0, NCHUNK // 8, group16, 0)

        pltpu.sync_copy(out_v, out_hbm.at[wid])

    return kern(cxt_idx, lbl_idx, cxt_table, lbl_table)


def _tc_loss(dots):
    def body(d_ref, o_ref):
        d = d_ref[...] * (1.0 / L)
        x = jnp.where(jnp.arange(2)[:, None] == 0, d, -d)
        sp = jnp.maximum(x, 0.0) + jnp.log1p(jnp.exp(-jnp.abs(x)))
        o_ref[0, 0] = jnp.sum(sp)

    out = pl.pallas_call(
        body,
        out_shape=jax.ShapeDtypeStruct((1, 1), jnp.float32),
        out_specs=pl.BlockSpec(memory_space=pltpu.SMEM),
    )(dots)
    return out[0, 0]


def kernel(l_cxt, r_cxt, l_lbl, r_lbl, cxt_table, lbl_table):
    cxt_idx = jnp.stack(
        [l_cxt.reshape(NW, NCHUNK, ROWS), r_cxt.reshape(NW, NCHUNK, ROWS)],
        axis=1).astype(jnp.int32)
    lbl_idx = jnp.stack(
        [(l_lbl - V).reshape(NW, BPW), (r_lbl - V).reshape(NW, BPW)],
        axis=1).astype(jnp.int32)
    dots = _sc_dots(cxt_idx, lbl_idx, cxt_table, lbl_table)
    dots = dots.transpose(1, 0, 2).reshape(2, B)
    return _tc_loss(dots)


# speedup vs baseline: 11.1708x; 1.1879x over previous
import functools

import jax
import jax.numpy as jnp
from jax import lax
from jax.experimental import pallas as pl
from jax.experimental.pallas import tpu as pltpu
from jax.experimental.pallas import tpu_sc as plsc

V = 100001
D = 64
B = 4096
L = 50
NC, NS = 2, 16
NW = NC * NS
BPW = B // NW
EPC = 2
NCHUNK = BPW // EPC
ROWS = EPC * L


def _sc_dots(cxt_idx, lbl_idx, cxt_table, lbl_table):
    mesh = plsc.VectorSubcoreMesh(core_axis_name="c", subcore_axis_name="s")

    @functools.partial(
        pl.kernel,
        out_type=jax.ShapeDtypeStruct((NW, 2, BPW), jnp.float32),
        mesh=mesh,
        scratch_types=[
            pltpu.VMEM((2, NCHUNK, ROWS), jnp.int32),
            pltpu.VMEM((2, BPW), jnp.int32),
            pltpu.VMEM((2, BPW, D), jnp.float32),
            pltpu.VMEM((4, ROWS, D), jnp.float32),
            pltpu.VMEM((2, BPW), jnp.float32),
            pltpu.SemaphoreType.DMA,
            pltpu.SemaphoreType.DMA,
            pltpu.SemaphoreType.DMA,
            pltpu.SemaphoreType.DMA,
            pltpu.SemaphoreType.DMA,
        ],
        compiler_params=pltpu.CompilerParams(use_tc_tiling_on_sc=False),
    )
    def kern(cxt_idx_hbm, lbl_idx_hbm, cxt_tab_hbm, lbl_tab_hbm, out_hbm,
             idx_v, lidx_v, lrows_v, buf_v, out_v,
             sem0, sem1, sem2, sem3, sem_l):
        wid = lax.axis_index("s") * NC + lax.axis_index("c")
        sems = (sem0, sem1, sem2, sem3)

        pltpu.sync_copy(cxt_idx_hbm.at[wid], idx_v)
        pltpu.sync_copy(lbl_idx_hbm.at[wid], lidx_v)
        pltpu.async_copy(lbl_tab_hbm.at[lidx_v.at[0]], lrows_v.at[0], sem_l).wait()
        pltpu.async_copy(lbl_tab_hbm.at[lidx_v.at[1]], lrows_v.at[1], sem_l).wait()

        lanes = lax.iota(jnp.int32, 16)
        for s in range(2):
            for b in range(4):
                pltpu.async_copy(
                    cxt_tab_hbm.at[idx_v.at[s, b]], buf_v.at[b], sems[b])

            def group16(g, _, s=s):
                dvec = jnp.zeros((16,), jnp.float32)
                for b8 in range(8):
                    chunk = 8 * g + b8
                    slot = b8 % 4
                    pltpu.make_async_copy(
                        cxt_tab_hbm.at[idx_v.at[s, slot]], buf_v.at[slot],
                        sems[slot]).wait()
                    for e in range(EPC):
                        bb = chunk * EPC + e
                        lane = b8 * EPC + e
                        lbl = [lrows_v[s, bb, pl.ds(16 * c, 16)]
                               for c in range(4)]

                        def row_acc(l, acc, e=e, slot=slot, lbl=lbl):
                            q = e * L + l
                            return tuple(
                                acc[c] + buf_v[slot, q, pl.ds(16 * c, 16)]
                                * lbl[c]
                                for c in range(4))

                        z = jnp.zeros((16,), jnp.float32)
                        a = lax.fori_loop(0, L, row_acc, (z, z, z, z),
                                          unroll=10)
                        tot = (a[0] + a[1]) + (a[2] + a[3])
                        for sh in (8, 4, 2, 1):
                            tot = tot + tot.at[lanes ^ sh].get(
                                mode="promise_in_bounds")
                        dvec = jnp.where(lanes == lane, tot, dvec)
                    @pl.when(chunk + 4 < NCHUNK)
                    def _(slot=slot, chunk=chunk, s=s):
                        pltpu.async_copy(
                            cxt_tab_hbm.at[idx_v.at[s, chunk + 4]],
                            buf_v.at[slot], sems[slot])
                out_v[s, pl.ds(g * 16, 16)] = dvec
                return 0

            lax.fori_loop(0, NCHUNK // 8, group16, 0)

        pltpu.sync_copy(out_v, out_hbm.at[wid])

    return kern(cxt_idx, lbl_idx, cxt_table, lbl_table)


def _tc_loss(dots):
    def body(d_ref, o_ref):
        d = d_ref[...] * (1.0 / L)
        x = jnp.where(jnp.arange(2)[:, None] == 0, d, -d)
        sp = jnp.maximum(x, 0.0) + jnp.log1p(jnp.exp(-jnp.abs(x)))
        o_ref[0, 0] = jnp.sum(sp)

    out = pl.pallas_call(
        body,
        out_shape=jax.ShapeDtypeStruct((1, 1), jnp.float32),
        out_specs=pl.BlockSpec(memory_space=pltpu.SMEM),
    )(dots)
    return out[0, 0]


def kernel(l_cxt, r_cxt, l_lbl, r_lbl, cxt_table, lbl_table):
    cxt_idx = jnp.stack(
        [l_cxt.reshape(NW, NCHUNK, ROWS), r_cxt.reshape(NW, NCHUNK, ROWS)],
        axis=1).astype(jnp.int32)
    lbl_idx = jnp.stack(
        [(l_lbl - V).reshape(NW, BPW), (r_lbl - V).reshape(NW, BPW)],
        axis=1).astype(jnp.int32)
    dots = _sc_dots(cxt_idx, lbl_idx, cxt_table, lbl_table)
    dots = dots.transpose(1, 0, 2).reshape(2, B)
    return _tc_loss(dots)
